# baseline (speedup 1.0000x reference)
import jax
import jax.numpy as jnp
from jax.experimental import pallas as pl
from jax.experimental.pallas import tpu as pltpu

NUM_EXPERTS = 64
BANDS = 16


def _body(maskT_hbm, routesT_hbm, imp_ref, load_ref, buf, sems):
    n_exp, n = buf.shape
    per = n // n_exp
    bre = n_exp // BANDS
    for b in range(BANDS):
        r = jax.lax.broadcasted_iota(jnp.int32, (bre, n), 0)
        c = jax.lax.broadcasted_iota(jnp.int32, (bre, n), 1)
        low = (r + b * bre) * per
        pat = ((c >= low) & (c < low + per)).astype(buf.dtype)
        buf[pl.ds(b * bre, bre), :] = pat
        s = jnp.sum(pat, axis=1)
        imp_ref[pl.ds(b * bre, bre)] = s
        load_ref[pl.ds(b * bre, bre)] = s
        for t, dst in enumerate((maskT_hbm, routesT_hbm)):
            pltpu.make_async_copy(
                buf.at[pl.ds(b * bre, bre), :],
                dst.at[pl.ds(b * bre, bre), :],
                sems.at[2 * b + t],
            ).start()
    for b in range(BANDS):
        for t, dst in enumerate((maskT_hbm, routesT_hbm)):
            pltpu.make_async_copy(
                buf.at[pl.ds(b * bre, bre), :],
                dst.at[pl.ds(b * bre, bre), :],
                sems.at[2 * b + t],
            ).wait()


def kernel(x):
    n = x.shape[0]
    assert n % NUM_EXPERTS == 0 and NUM_EXPERTS % BANDS == 0
    dt = x.dtype
    maskT, routesT, imp, load = pl.pallas_call(
        _body,
        out_shape=(
            jax.ShapeDtypeStruct((NUM_EXPERTS, n), dt),
            jax.ShapeDtypeStruct((NUM_EXPERTS, n), dt),
            jax.ShapeDtypeStruct((NUM_EXPERTS,), dt),
            jax.ShapeDtypeStruct((NUM_EXPERTS,), dt),
        ),
        out_specs=(
            pl.BlockSpec(memory_space=pltpu.MemorySpace.HBM),
            pl.BlockSpec(memory_space=pltpu.MemorySpace.HBM),
            pl.BlockSpec(memory_space=pltpu.MemorySpace.VMEM),
            pl.BlockSpec(memory_space=pltpu.MemorySpace.VMEM),
        ),
        scratch_shapes=[
            pltpu.VMEM((NUM_EXPERTS, n), dt),
            pltpu.SemaphoreType.DMA((2 * BANDS,)),
        ],
    )()
    return (maskT.T, routesT.T, imp, load)


# speedup vs baseline: 1.5169x; 1.5169x over previous
import jax
import jax.numpy as jnp
from jax.experimental import pallas as pl
from jax.experimental.pallas import tpu as pltpu

NUM_EXPERTS = 64
BANDS = 4


def _body(maskT_hbm, routesT_hbm, imp_ref, load_ref, buf, sems):
    n_exp, n = buf.shape
    per = n // n_exp
    bre = n_exp // BANDS
    for b in range(BANDS):
        r = jax.lax.broadcasted_iota(jnp.int32, (bre, n), 0)
        c = jax.lax.broadcasted_iota(jnp.int32, (bre, n), 1)
        low = (r + b * bre) * per
        pat = ((c >= low) & (c < low + per)).astype(buf.dtype)
        buf[pl.ds(b * bre, bre), :] = pat
        s = jnp.sum(pat, axis=1)
        imp_ref[pl.ds(b * bre, bre)] = s
        load_ref[pl.ds(b * bre, bre)] = s
        for t, dst in enumerate((maskT_hbm, routesT_hbm)):
            pltpu.make_async_copy(
                buf.at[pl.ds(b * bre, bre), :],
                dst.at[pl.ds(b * bre, bre), :],
                sems.at[2 * b + t],
            ).start()
    for b in range(BANDS):
        for t, dst in enumerate((maskT_hbm, routesT_hbm)):
            pltpu.make_async_copy(
                buf.at[pl.ds(b * bre, bre), :],
                dst.at[pl.ds(b * bre, bre), :],
                sems.at[2 * b + t],
            ).wait()


def kernel(x):
    n = x.shape[0]
    assert n % NUM_EXPERTS == 0 and NUM_EXPERTS % BANDS == 0
    dt = x.dtype
    maskT, routesT, imp, load = pl.pallas_call(
        _body,
        out_shape=(
            jax.ShapeDtypeStruct((NUM_EXPERTS, n), dt),
            jax.ShapeDtypeStruct((NUM_EXPERTS, n), dt),
            jax.ShapeDtypeStruct((NUM_EXPERTS,), dt),
            jax.ShapeDtypeStruct((NUM_EXPERTS,), dt),
        ),
        out_specs=(
            pl.BlockSpec(memory_space=pltpu.MemorySpace.HBM),
            pl.BlockSpec(memory_space=pltpu.MemorySpace.HBM),
            pl.BlockSpec(memory_space=pltpu.MemorySpace.VMEM),
            pl.BlockSpec(memory_space=pltpu.MemorySpace.VMEM),
        ),
        scratch_shapes=[
            pltpu.VMEM((NUM_EXPERTS, n), dt),
            pltpu.SemaphoreType.DMA((2 * BANDS,)),
        ],
    )()
    return (maskT.T, routesT.T, imp, load)
